# baseline (speedup 1.0000x reference)
import jax
import jax.numpy as jnp
from jax import lax
from jax.experimental import pallas as pl
from jax.experimental.pallas import tpu as pltpu
from jax.experimental.pallas import tpu_sc as plsc

INPUT = 1024
HIDDEN = 4096
OUTPUT = 1024
E = 8
N = 2048

E_SC = 2
E_TC = E - E_SC
HCHUNK = 1024
NC = HIDDEN // HCHUNK
T_TC = E_TC * NC

NW = 32
TPE = NW // E_SC
HS_T = HIDDEN // TPE
IB = 64
NIB = INPUT // IB
W2B = 32
NOB = HS_T // W2B

_SQRT_HALF = 0.7071067811865476
_L16 = 16


def _router_body(x_ref, wr_ref, br_ref, wcomb_ref, xfirst_ref):
    x = x_ref[...]
    logits = jnp.dot(x, wr_ref[...], preferred_element_type=jnp.float32)
    logits = logits + br_ref[...]

    iota_e = lax.broadcasted_iota(jnp.int32, (N, E), 1)
    m1 = jnp.max(logits, axis=-1, keepdims=True)
    a1 = jnp.min(jnp.where(logits == m1, iota_e, E), axis=-1, keepdims=True)
    masked = jnp.where(iota_e == a1, -jnp.inf, logits)
    m2 = jnp.max(masked, axis=-1, keepdims=True)
    a2 = jnp.min(jnp.where(masked == m2, iota_e, E), axis=-1, keepdims=True)

    r = jnp.exp(m2 - m1)
    denom = 1.0 + r
    p1 = 1.0 / denom
    p2 = r / denom

    sel1 = iota_e == a1
    sel2 = iota_e == a2
    wcomb_ref[...] = jnp.where(sel1, p1, 0.0) + jnp.where(sel2, p2, 0.0)

    sel = sel1 | sel2
    iota_n = lax.broadcasted_iota(jnp.int32, (N, E), 0)
    fi = jnp.min(jnp.where(sel, iota_n, N), axis=0, keepdims=True)
    onehot = (iota_n == fi).astype(jnp.float32)
    xfirst_ref[...] = lax.dot_general(
        onehot, x, (((0,), (0,)), ((), ())),
        preferred_element_type=jnp.float32)


def _tc_ffn_body(xfirst_ref, w1_ref, b1_ref, w2_ref, b2_ref, f_ref, f_s):
    t = pl.program_id(0)
    e = t // NC
    c = t % NC

    @pl.when(t == 0)
    def _():
        f_s[...] = b2_ref[:, 0, :]

    iota_row = lax.broadcasted_iota(jnp.int32, (1, E), 1)
    oh_e = (iota_row == e).astype(jnp.float32)
    xr = jnp.dot(oh_e, xfirst_ref[...], preferred_element_type=jnp.float32)

    h = jnp.dot(xr, w1_ref[0], preferred_element_type=jnp.float32)
    h = h + b1_ref[0]
    g = 0.5 * h * (1.0 + lax.erf(h * _SQRT_HALF))
    part = jnp.dot(g, w2_ref[0], preferred_element_type=jnp.float32)

    rmask = (lax.broadcasted_iota(jnp.int32, (E, 1), 0) == e).astype(jnp.float32)
    f_s[...] += rmask * part

    @pl.when(t == T_TC - 1)
    def _():
        f_ref[...] = f_s[...]


def _bcast_elem(ref, i):
    idx = jnp.full((_L16,), i, jnp.int32)
    return plsc.load_gather(ref, [idx])


def _gelu16(hk):
    z = jnp.abs(hk) * _SQRT_HALF
    t = 1.0 / (1.0 + 0.3275911 * z)
    poly = ((((1.061405429 * t - 1.453152027) * t + 1.421413741) * t
             - 0.284496736) * t + 0.254829592) * t
    erf_abs = 1.0 - poly * jnp.exp(-z * z)
    erf_signed = jnp.where(hk >= 0, erf_abs, -erf_abs)
    return 0.5 * hk * (1.0 + erf_signed)


def _sc_ffn_kernel(xfirst_hbm, w1_hbm, b1_hbm, w2_hbm, fsc_hbm,
                   xr_v, w1buf, b1buf, gbuf, w2buf, facc_v, sem):
    cid = lax.axis_index("c")
    sid = lax.axis_index("s")
    wid = sid * 2 + cid
    e = E_TC + wid // TPE
    sl = wid % TPE
    h0 = sl * HS_T

    pltpu.sync_copy(xfirst_hbm.at[e], xr_v)
    pltpu.sync_copy(b1_hbm.at[e, pl.ds(h0, HS_T)], b1buf)

    h_init = tuple(b1buf[pl.ds(k * _L16, _L16)] for k in range(HS_T // _L16))

    def h_block(ib, h):
        d0 = ib * IB
        copy = pltpu.async_copy(
            w1_hbm.at[e, pl.ds(d0, IB), pl.ds(h0, HS_T)], w1buf, sem)
        copy.wait()
        h = list(h)
        for row in range(IB):
            w = _bcast_elem(xr_v, d0 + row)
            for k in range(HS_T // _L16):
                h[k] = h[k] + w * w1buf[row, pl.ds(k * _L16, _L16)]
        return tuple(h)

    h_fin = lax.fori_loop(0, NIB, h_block, h_init, unroll=False)

    for k in range(HS_T // _L16):
        gbuf[pl.ds(k * _L16, _L16)] = _gelu16(h_fin[k])

    zero = jnp.zeros((_L16,), jnp.float32)
    for m in range(OUTPUT // _L16):
        facc_v[pl.ds(m * _L16, _L16)] = zero

    def o_block(ob, carry):
        copy = pltpu.async_copy(
            w2_hbm.at[e, pl.ds(h0 + ob * W2B, W2B), :], w2buf, sem)
        copy.wait()
        for half in range(2):
            acc = [zero] * (OUTPUT // _L16 // 2)
            for row in range(W2B):
                gv = _bcast_elem(gbuf, ob * W2B + row)
                for m in range(OUTPUT // _L16 // 2):
                    col = half * (OUTPUT // 2) + m * _L16
                    acc[m] = acc[m] + gv * w2buf[row, pl.ds(col, _L16)]
            for m in range(OUTPUT // _L16 // 2):
                col = half * (OUTPUT // 2) + m * _L16
                facc_v[pl.ds(col, _L16)] = facc_v[pl.ds(col, _L16)] + acc[m]
        return carry

    lax.fori_loop(0, NOB, o_block, 0, unroll=False)

    pltpu.sync_copy(facc_v, fsc_hbm.at[wid])


def _sc_ffn(xfirst, W1, b1, W2):
    mesh = plsc.VectorSubcoreMesh(core_axis_name="c", subcore_axis_name="s")
    return pl.kernel(
        _sc_ffn_kernel,
        out_type=jax.ShapeDtypeStruct((NW, OUTPUT), jnp.float32),
        mesh=mesh,
        compiler_params=pltpu.CompilerParams(needs_layout_passes=False),
        scratch_types=[
            pltpu.VMEM((INPUT,), jnp.float32),
            pltpu.VMEM((IB, HS_T), jnp.float32),
            pltpu.VMEM((HS_T,), jnp.float32),
            pltpu.VMEM((HS_T,), jnp.float32),
            pltpu.VMEM((W2B, OUTPUT), jnp.float32),
            pltpu.VMEM((OUTPUT,), jnp.float32),
            pltpu.SemaphoreType.DMA,
        ],
    )(xfirst, W1, b1, W2)


def _combine_body(wcomb_ref, ftc_ref, fsc_ref, gamma_ref, beta_ref, out_ref):
    iota_e = lax.broadcasted_iota(jnp.int32, (E, NW), 0)
    iota_w = lax.broadcasted_iota(jnp.int32, (E, NW), 1)
    sel = (iota_e == E_TC + iota_w // TPE).astype(jnp.float32)
    f = ftc_ref[...] + jnp.dot(sel, fsc_ref[...],
                               preferred_element_type=jnp.float32)
    pre = jnp.dot(wcomb_ref[...], f,
                  preferred_element_type=jnp.float32)
    mean = jnp.mean(pre, axis=-1, keepdims=True)
    d = pre - mean
    var = jnp.mean(d * d, axis=-1, keepdims=True)
    inv = lax.rsqrt(var + 1e-5)
    out_ref[...] = d * inv * gamma_ref[...] + beta_ref[...]


@jax.jit
def kernel(x, Wr, br, W1, b1, W2, b2, gamma, beta):
    Bc, S, D = x.shape
    xf = x.reshape(Bc * S, D)

    wcomb, xfirst = pl.pallas_call(
        _router_body,
        out_shape=(
            jax.ShapeDtypeStruct((N, E), jnp.float32),
            jax.ShapeDtypeStruct((E, INPUT), jnp.float32),
        ),
    )(xf, Wr, br.reshape(1, E))

    f_sc = _sc_ffn(xfirst, W1, b1, W2)

    f_tc = pl.pallas_call(
        _tc_ffn_body,
        grid=(T_TC,),
        in_specs=[
            pl.BlockSpec((E, INPUT), lambda t: (0, 0)),
            pl.BlockSpec((1, INPUT, HCHUNK), lambda t: (t // NC, 0, t % NC)),
            pl.BlockSpec((1, 1, HCHUNK), lambda t: (t // NC, 0, t % NC)),
            pl.BlockSpec((1, HCHUNK, OUTPUT), lambda t: (t // NC, t % NC, 0)),
            pl.BlockSpec((E, 1, OUTPUT), lambda t: (0, 0, 0)),
        ],
        out_specs=pl.BlockSpec((E, OUTPUT), lambda t: (0, 0)),
        out_shape=jax.ShapeDtypeStruct((E, OUTPUT), jnp.float32),
        scratch_shapes=[pltpu.VMEM((E, OUTPUT), jnp.float32)],
    )(xfirst, W1, b1.reshape(E, 1, HIDDEN), W2, b2.reshape(E, 1, OUTPUT))

    out = pl.pallas_call(
        _combine_body,
        out_shape=jax.ShapeDtypeStruct((N, OUTPUT), jnp.float32),
    )(wcomb, f_tc, f_sc, gamma.reshape(1, OUTPUT), beta.reshape(1, OUTPUT))

    return out.reshape(Bc, S, OUTPUT)


# speedup vs baseline: 2.2744x; 2.2744x over previous
import jax
import jax.numpy as jnp
from jax import lax
from jax.experimental import pallas as pl
from jax.experimental.pallas import tpu as pltpu

INPUT = 1024
HIDDEN = 4096
OUTPUT = 1024
E = 8
N = 2048
HCHUNK = 1024
NC = HIDDEN // HCHUNK
T = E * NC

_SQRT_HALF = 0.7071067811865476


def _fused_body(x_ref, wr_ref, br_ref, w1_ref, b1_ref, w2_ref, b2_ref,
                gamma_ref, beta_ref, out_ref, wcomb_s, xfirst_s, f_s):
    t = pl.program_id(0)
    e = t // NC
    c = t % NC

    @pl.when(t == 0)
    def _router():
        x = x_ref[...]
        logits = jnp.dot(x, wr_ref[...], preferred_element_type=jnp.float32)
        logits = logits + br_ref[...]

        iota_e = lax.broadcasted_iota(jnp.int32, (N, E), 1)
        m1 = jnp.max(logits, axis=-1, keepdims=True)
        a1 = jnp.min(jnp.where(logits == m1, iota_e, E), axis=-1, keepdims=True)
        masked = jnp.where(iota_e == a1, -jnp.inf, logits)
        m2 = jnp.max(masked, axis=-1, keepdims=True)
        a2 = jnp.min(jnp.where(masked == m2, iota_e, E), axis=-1, keepdims=True)

        r = jnp.exp(m2 - m1)
        denom = 1.0 + r
        p1 = 1.0 / denom
        p2 = r / denom

        sel1 = iota_e == a1
        sel2 = iota_e == a2
        wcomb_s[...] = jnp.where(sel1, p1, 0.0) + jnp.where(sel2, p2, 0.0)

        sel = sel1 | sel2
        iota_n = lax.broadcasted_iota(jnp.int32, (N, E), 0)
        fi = jnp.min(jnp.where(sel, iota_n, N), axis=0, keepdims=True)
        onehot = (iota_n == fi).astype(jnp.float32)
        xfirst_s[...] = lax.dot_general(
            onehot, x, (((0,), (0,)), ((), ())),
            preferred_element_type=jnp.float32)
        f_s[...] = b2_ref[:, 0, :]

    iota_row = lax.broadcasted_iota(jnp.int32, (1, E), 1)
    oh_e = (iota_row == e).astype(jnp.float32)
    xr = jnp.dot(oh_e, xfirst_s[...], preferred_element_type=jnp.float32)

    h = jnp.dot(xr, w1_ref[0], preferred_element_type=jnp.float32)
    h = h + b1_ref[0]
    g = 0.5 * h * (1.0 + lax.erf(h * _SQRT_HALF))
    part = jnp.dot(g, w2_ref[0], preferred_element_type=jnp.float32)

    rmask = (lax.broadcasted_iota(jnp.int32, (E, 1), 0) == e).astype(jnp.float32)
    f_s[...] += rmask * part

    @pl.when(t == T - 1)
    def _combine():
        pre = jnp.dot(wcomb_s[...], f_s[...],
                      preferred_element_type=jnp.float32)
        mean = jnp.mean(pre, axis=-1, keepdims=True)
        d = pre - mean
        var = jnp.mean(d * d, axis=-1, keepdims=True)
        inv = lax.rsqrt(var + 1e-5)
        out_ref[...] = d * inv * gamma_ref[...] + beta_ref[...]


@jax.jit
def kernel(x, Wr, br, W1, b1, W2, b2, gamma, beta):
    Bc, S, D = x.shape
    xf = x.reshape(Bc * S, D)

    out = pl.pallas_call(
        _fused_body,
        grid=(T,),
        in_specs=[
            pl.BlockSpec((N, INPUT), lambda t: (0, 0)),
            pl.BlockSpec((INPUT, E), lambda t: (0, 0)),
            pl.BlockSpec((1, E), lambda t: (0, 0)),
            pl.BlockSpec((1, INPUT, HCHUNK), lambda t: (t // NC, 0, t % NC)),
            pl.BlockSpec((1, 1, HCHUNK), lambda t: (t // NC, 0, t % NC)),
            pl.BlockSpec((1, HCHUNK, OUTPUT), lambda t: (t // NC, t % NC, 0)),
            pl.BlockSpec((E, 1, OUTPUT), lambda t: (0, 0, 0)),
            pl.BlockSpec((1, OUTPUT), lambda t: (0, 0)),
            pl.BlockSpec((1, OUTPUT), lambda t: (0, 0)),
        ],
        out_specs=pl.BlockSpec((N, OUTPUT), lambda t: (0, 0)),
        out_shape=jax.ShapeDtypeStruct((N, OUTPUT), jnp.float32),
        scratch_shapes=[
            pltpu.VMEM((N, E), jnp.float32),
            pltpu.VMEM((E, INPUT), jnp.float32),
            pltpu.VMEM((E, OUTPUT), jnp.float32),
        ],
    )(xf, Wr, br.reshape(1, E), W1, b1.reshape(E, 1, HIDDEN),
      W2, b2.reshape(E, 1, OUTPUT), gamma.reshape(1, OUTPUT),
      beta.reshape(1, OUTPUT))

    return out.reshape(Bc, S, OUTPUT)


# speedup vs baseline: 2.2902x; 1.0069x over previous
import jax
import jax.numpy as jnp
from jax import lax
from jax.experimental import pallas as pl
from jax.experimental.pallas import tpu as pltpu

INPUT = 1024
HIDDEN = 4096
OUTPUT = 1024
E = 8
N = 2048
HCHUNK = 1024
NC = HIDDEN // HCHUNK
T = E * NC
NS = 4
NBLK = N // NS

_SQRT_HALF = 0.7071067811865476


def _fused_body(x_ref, wr_ref, br_ref, w1_ref, b1_ref, w2_ref, b2_ref,
                gamma_ref, beta_ref, out_ref, wcomb_s, xfirst_s, f_s):
    t = pl.program_id(0)
    e = jnp.minimum(t, T - 1) // NC

    @pl.when(t == 0)
    def _router():
        x = x_ref[...]
        logits = jnp.dot(x, wr_ref[...], preferred_element_type=jnp.float32)
        logits = logits + br_ref[...]

        iota_e = lax.broadcasted_iota(jnp.int32, (N, E), 1)
        m1 = jnp.max(logits, axis=-1, keepdims=True)
        a1 = jnp.min(jnp.where(logits == m1, iota_e, E), axis=-1, keepdims=True)
        masked = jnp.where(iota_e == a1, -jnp.inf, logits)
        m2 = jnp.max(masked, axis=-1, keepdims=True)
        a2 = jnp.min(jnp.where(masked == m2, iota_e, E), axis=-1, keepdims=True)

        r = jnp.exp(m2 - m1)
        denom = 1.0 + r
        p1 = 1.0 / denom
        p2 = r / denom

        sel1 = iota_e == a1
        sel2 = iota_e == a2
        wcomb_s[...] = jnp.where(sel1, p1, 0.0) + jnp.where(sel2, p2, 0.0)

        sel = sel1 | sel2
        iota_n = lax.broadcasted_iota(jnp.int32, (N, E), 0)
        fi = jnp.min(jnp.where(sel, iota_n, N), axis=0, keepdims=True)
        onehot = (iota_n == fi).astype(jnp.float32)
        xfirst_s[...] = lax.dot_general(
            onehot, x, (((0,), (0,)), ((), ())),
            preferred_element_type=jnp.float32)
        f_s[...] = b2_ref[:, 0, :]

    @pl.when(t < T)
    def _ffn():
        iota_row = lax.broadcasted_iota(jnp.int32, (1, E), 1)
        oh_e = (iota_row == e).astype(jnp.float32)
        xr = jnp.dot(oh_e, xfirst_s[...],
                     preferred_element_type=jnp.float32)

        h = jnp.dot(xr, w1_ref[0], preferred_element_type=jnp.float32)
        h = h + b1_ref[0]
        g = 0.5 * h * (1.0 + lax.erf(h * _SQRT_HALF))
        part = jnp.dot(g, w2_ref[0], preferred_element_type=jnp.float32)

        rmask = (lax.broadcasted_iota(jnp.int32, (E, 1), 0) == e)
        f_s[...] += rmask.astype(jnp.float32) * part

    @pl.when(t >= T)
    def _combine():
        blk = t - T
        rows = wcomb_s[pl.ds(blk * NBLK, NBLK), :]
        pre = jnp.dot(rows, f_s[...],
                      preferred_element_type=jnp.float32)
        mean = jnp.mean(pre, axis=-1, keepdims=True)
        d = pre - mean
        var = jnp.mean(d * d, axis=-1, keepdims=True)
        inv = lax.rsqrt(var + 1e-5)
        out_ref[...] = d * inv * gamma_ref[...] + beta_ref[...]


@jax.jit
def kernel(x, Wr, br, W1, b1, W2, b2, gamma, beta):
    Bc, S, D = x.shape
    xf = x.reshape(Bc * S, D)

    def wmap(t):
        tc = jnp.minimum(t, T - 1)
        return tc // NC, tc % NC

    out = pl.pallas_call(
        _fused_body,
        grid=(T + NS,),
        in_specs=[
            pl.BlockSpec((N, INPUT), lambda t: (0, 0)),
            pl.BlockSpec((INPUT, E), lambda t: (0, 0)),
            pl.BlockSpec((1, E), lambda t: (0, 0)),
            pl.BlockSpec((1, INPUT, HCHUNK),
                         lambda t: (wmap(t)[0], 0, wmap(t)[1])),
            pl.BlockSpec((1, 1, HCHUNK),
                         lambda t: (wmap(t)[0], 0, wmap(t)[1])),
            pl.BlockSpec((1, HCHUNK, OUTPUT),
                         lambda t: (wmap(t)[0], wmap(t)[1], 0)),
            pl.BlockSpec((E, 1, OUTPUT), lambda t: (0, 0, 0)),
            pl.BlockSpec((1, OUTPUT), lambda t: (0, 0)),
            pl.BlockSpec((1, OUTPUT), lambda t: (0, 0)),
        ],
        out_specs=pl.BlockSpec(
            (NBLK, OUTPUT),
            lambda t: (jnp.clip(t - T, 0, NS - 1), 0)),
        out_shape=jax.ShapeDtypeStruct((N, OUTPUT), jnp.float32),
        scratch_shapes=[
            pltpu.VMEM((N, E), jnp.float32),
            pltpu.VMEM((E, INPUT), jnp.float32),
            pltpu.VMEM((E, OUTPUT), jnp.float32),
        ],
    )(xf, Wr, br.reshape(1, E), W1, b1.reshape(E, 1, HIDDEN),
      W2, b2.reshape(E, 1, OUTPUT), gamma.reshape(1, OUTPUT),
      beta.reshape(1, OUTPUT))

    return out.reshape(Bc, S, OUTPUT)
